# baseline (speedup 1.0000x reference)
import functools

import jax
import jax.numpy as jnp
from jax import lax
from jax.experimental import pallas as pl
from jax.experimental.pallas import tpu as pltpu
from jax.experimental.pallas import tpu_sc as plsc

B = 4096
BX = 64
BY = 64
NKEY = BX * BY
NWORDS = NKEY * B

NC = 2
NS = 16
NW = NC * NS
KPW = NKEY // NW
WPW = KPW * B

NBUF = 4
CW = 16384
NCH = WPW // CW
NGRP = NCH // NBUF

_mesh = plsc.VectorSubcoreMesh(
    core_axis_name="c", subcore_axis_name="s", num_cores=NC, num_subcores=NS
)


@functools.partial(
    pl.kernel,
    out_type=jax.ShapeDtypeStruct((NWORDS,), jnp.int32),
    mesh=_mesh,
    compiler_params=pltpu.CompilerParams(needs_layout_passes=False),
    scratch_types=[
        pltpu.VMEM((B * 2,), jnp.int32),
        pltpu.VMEM((B + 16,), jnp.int32),
        *[pltpu.VMEM((CW,), jnp.int32) for _ in range(NBUF)],
        *[pltpu.SemaphoreType.DMA for _ in range(NBUF)],
        pltpu.SemaphoreType.DMA,
    ],
)
def _onehot2d_sc(xy_hbm, out_hbm, xy_v, ml_v, *bufsems):
    bufs = bufsems[:NBUF]
    sems = bufsems[NBUF:2 * NBUF]
    lsem = bufsems[2 * NBUF]
    wid = lax.axis_index("s") * NC + lax.axis_index("c")
    klo = wid * KPW
    wbase = wid * WPW

    iota = lax.iota(jnp.int32, 16)
    zv = jnp.zeros((16,), jnp.int32)
    ones = jnp.full((16,), 1, jnp.int32)

    stage = pltpu.async_copy(xy_hbm, xy_v, lsem)

    def _zero(buf):
        def body(i, c):
            for u in range(8):
                buf[pl.ds(i * 128 + u * 16, 16)] = zv
            return c
        lax.fori_loop(0, CW // 128, body, 0)

    _zero(bufs[0])
    _zero(bufs[1])
    stage.wait()

    def _scan1(i, off):
        base = i * 16 + (i // 8) * 128
        xv = xy_v[pl.ds(base, 16)]
        yv = xy_v[pl.ds(base + 128, 16)]
        key = xv * BY + yv
        m = (key >= klo) & (key < klo + KPW)
        mw = (key - klo) * B + iota + i * 16
        plsc.store_compressed(ml_v.at[pl.ds(off, 16)], mw, mask=m)
        cnt = plsc.all_reduce_population_count(m)
        return off + cnt[0]

    def _scan(i, off):
        off = _scan1(2 * i, off)
        return _scan1(2 * i + 1, off)

    nmatch = lax.fori_loop(0, B // 32, _scan, 0)
    ml_v[pl.ds(nmatch, 16)] = jnp.full((16,), -1, jnp.int32)
    ngrp = (nmatch + 15) // 16

    def _paint(buf, lo, val):
        def body(i, c):
            mv = ml_v[pl.ds(i * 16, 16)]
            m = (mv >= lo) & (mv < lo + CW)
            idx = lax.select(m, mv - lo, zv)
            plsc.store_scatter(buf, [idx], val, mask=m)
            return c
        lax.fori_loop(0, ngrp, body, 0)

    def _fire(buf, lo, sem):
        pltpu.async_copy(buf, out_hbm.at[pl.ds(wbase + lo, CW)], sem)

    def _drain(buf, sem):
        pltpu.make_async_copy(buf, out_hbm.at[pl.ds(wbase, CW)], sem).wait()

    for u in range(NBUF):
        if u >= 2:
            _zero(bufs[u])
        _paint(bufs[u], u * CW, ones)
        _fire(bufs[u], u * CW, sems[u])

    for c in range(NBUF, NCH):
        u = c % NBUF
        lo = c * CW
        _drain(bufs[u], sems[u])
        _paint(bufs[u], lo - NBUF * CW, zv)
        _paint(bufs[u], lo, ones)
        _fire(bufs[u], lo, sems[u])

    for u in range(NBUF):
        _drain(bufs[u], sems[u])


def kernel(loc):
    xy = loc.reshape(32, 128, 2).transpose(0, 2, 1).reshape(-1)
    flat = _onehot2d_sc(xy)
    return flat.reshape(BX, BY, B, 1).transpose(2, 0, 1, 3)


# speedup vs baseline: 1.0394x; 1.0394x over previous
import functools

import jax
import jax.numpy as jnp
from jax import lax
from jax.experimental import pallas as pl
from jax.experimental.pallas import tpu as pltpu
from jax.experimental.pallas import tpu_sc as plsc

B = 4096
BX = 64
BY = 64
NKEY = BX * BY
NWORDS = NKEY * B

NC = 2
NS = 16
NW = NC * NS
KPW = NKEY // NW
WPW = KPW * B

NBUF = 4
CW = 16384
NCH = WPW // CW
NGRP = NCH // NBUF

_mesh = plsc.VectorSubcoreMesh(
    core_axis_name="c", subcore_axis_name="s", num_cores=NC, num_subcores=NS
)


@functools.partial(
    pl.kernel,
    out_type=jax.ShapeDtypeStruct((NWORDS,), jnp.int32),
    mesh=_mesh,
    compiler_params=pltpu.CompilerParams(needs_layout_passes=False),
    scratch_types=[
        pltpu.VMEM((B * 2,), jnp.int32),
        pltpu.VMEM((B + 16,), jnp.int32),
        *[pltpu.VMEM((CW,), jnp.int32) for _ in range(NBUF)],
        *[pltpu.SemaphoreType.DMA for _ in range(NBUF)],
        pltpu.SemaphoreType.DMA,
    ],
)
def _onehot2d_sc(xy_hbm, out_hbm, xy_v, ml_v, *bufsems):
    bufs = bufsems[:NBUF]
    sems = bufsems[NBUF:2 * NBUF]
    lsem = bufsems[2 * NBUF]
    wid = lax.axis_index("s") * NC + lax.axis_index("c")
    klo = wid * KPW
    wbase = wid * WPW

    iota = lax.iota(jnp.int32, 16)
    zv = jnp.zeros((16,), jnp.int32)
    ones = jnp.full((16,), 1, jnp.int32)

    stage = pltpu.async_copy(xy_hbm, xy_v, lsem)

    def _zero(buf):
        def body(i, c):
            for u in range(8):
                buf[pl.ds(i * 128 + u * 16, 16)] = zv
            return c
        lax.fori_loop(0, CW // 128, body, 0)

    _zero(bufs[0])
    _zero(bufs[1])
    stage.wait()

    def _scan1(i, off):
        base = i * 16 + (i // 8) * 128
        xv = xy_v[pl.ds(base, 16)]
        yv = xy_v[pl.ds(base + 128, 16)]
        key = xv * BY + yv
        m = (key >= klo) & (key < klo + KPW)
        mw = (key - klo) * B + iota + i * 16
        plsc.store_compressed(ml_v.at[pl.ds(off, 16)], mw, mask=m)
        cnt = plsc.all_reduce_population_count(m)
        return off + cnt[0]

    def _scan(i, off):
        off = _scan1(2 * i, off)
        return _scan1(2 * i + 1, off)

    nmatch = lax.fori_loop(0, B // 32, _scan, 0)
    ml_v[pl.ds(nmatch, 16)] = jnp.full((16,), -1, jnp.int32)
    ngrp = (nmatch + 15) // 16

    _zero(bufs[2])
    _zero(bufs[3])

    def _paint(buf, lo):
        def body(i, c):
            mv = ml_v[pl.ds(i * 16, 16)]
            m0 = (mv >= lo - NBUF * CW) & (mv < lo - (NBUF - 1) * CW)
            i0 = lax.select(m0, mv - (lo - NBUF * CW), zv)
            plsc.store_scatter(buf, [i0], zv, mask=m0)
            m1 = (mv >= lo) & (mv < lo + CW)
            i1 = lax.select(m1, mv - lo, zv)
            plsc.store_scatter(buf, [i1], ones, mask=m1)
            return c
        lax.fori_loop(0, ngrp, body, 0)

    def _drain(buf, sem):
        pltpu.make_async_copy(buf, out_hbm.at[pl.ds(wbase, CW)], sem).wait()

    def _group(g, c):
        for u in range(NBUF):
            lo = (g * NBUF + u) * CW

            @pl.when(g > 0)
            def _():
                _drain(bufs[u], sems[u])

            _paint(bufs[u], lo)
            pltpu.async_copy(bufs[u], out_hbm.at[pl.ds(wbase + lo, CW)], sems[u])
        return c

    lax.fori_loop(0, NGRP, _group, 0)

    for u in range(NBUF):
        _drain(bufs[u], sems[u])


def kernel(loc):
    xy = loc.reshape(32, 128, 2).transpose(0, 2, 1).reshape(-1)
    flat = _onehot2d_sc(xy)
    return flat.reshape(BX, BY, B, 1).transpose(2, 0, 1, 3)


# speedup vs baseline: 1.0659x; 1.0255x over previous
import functools

import jax
import jax.numpy as jnp
from jax import lax
from jax.experimental import pallas as pl
from jax.experimental.pallas import tpu as pltpu
from jax.experimental.pallas import tpu_sc as plsc

B = 4096
BX = 64
BY = 64
NKEY = BX * BY
NWORDS = NKEY * B

NC = 2
NS = 16
NW = NC * NS
KPW = NKEY // NW
WPW = KPW * B

NBUF = 4
CW = 16384
NCH = WPW // CW
NGRP = NCH // NBUF

_mesh = plsc.VectorSubcoreMesh(
    core_axis_name="c", subcore_axis_name="s", num_cores=NC, num_subcores=NS
)


@functools.partial(
    pl.kernel,
    out_type=jax.ShapeDtypeStruct((NWORDS,), jnp.int32),
    mesh=_mesh,
    compiler_params=pltpu.CompilerParams(needs_layout_passes=False),
    scratch_types=[
        pltpu.VMEM((B * 2,), jnp.int32),
        pltpu.VMEM((B + 16,), jnp.int32),
        *[pltpu.VMEM((CW,), jnp.int32) for _ in range(NBUF)],
        *[pltpu.SemaphoreType.DMA for _ in range(NBUF)],
        pltpu.SemaphoreType.DMA,
    ],
)
def _onehot2d_sc(xy_hbm, out_hbm, xy_v, ml_v, *bufsems):
    bufs = bufsems[:NBUF]
    sems = bufsems[NBUF:2 * NBUF]
    lsem = bufsems[2 * NBUF]
    wid = lax.axis_index("s") * NC + lax.axis_index("c")
    klo = wid * KPW
    wbase = wid * WPW

    iota = lax.iota(jnp.int32, 16)
    zv = jnp.zeros((16,), jnp.int32)
    ones = jnp.full((16,), 1, jnp.int32)

    stage = pltpu.async_copy(xy_hbm, xy_v, lsem)

    def _zero(buf):
        def body(i, c):
            for u in range(8):
                buf[pl.ds(i * 128 + u * 16, 16)] = zv
            return c
        lax.fori_loop(0, CW // 128, body, 0)

    _zero(bufs[0])
    _zero(bufs[1])
    stage.wait()

    def _scan1(i, off):
        base = i * 16 + (i // 8) * 128
        xv = xy_v[pl.ds(base, 16)]
        yv = xy_v[pl.ds(base + 128, 16)]
        key = xv * BY + yv
        m = (key >= klo) & (key < klo + KPW)
        mw = (key - klo) * B + iota + i * 16
        plsc.store_compressed(ml_v.at[pl.ds(off, 16)], mw, mask=m)
        cnt = plsc.all_reduce_population_count(m)
        return off + cnt[0]

    def _scan(i, off):
        off = _scan1(2 * i, off)
        return _scan1(2 * i + 1, off)

    nmatch = lax.fori_loop(0, B // 32, _scan, 0)
    ml_v[pl.ds(nmatch, 16)] = jnp.full((16,), -1, jnp.int32)
    ngrp = (nmatch + 15) // 16

    def _paint(buf, lo):
        def body(i, c):
            mv = ml_v[pl.ds(i * 16, 16)]
            m0 = (mv >= lo - NBUF * CW) & (mv < lo - (NBUF - 1) * CW)
            i0 = lax.select(m0, mv - (lo - NBUF * CW), zv)
            plsc.store_scatter(buf, [i0], zv, mask=m0)
            m1 = (mv >= lo) & (mv < lo + CW)
            i1 = lax.select(m1, mv - lo, zv)
            plsc.store_scatter(buf, [i1], ones, mask=m1)
            return c
        lax.fori_loop(0, ngrp, body, 0)

    def _drain(buf, sem):
        pltpu.make_async_copy(buf, out_hbm.at[pl.ds(wbase, CW)], sem).wait()

    for u in range(NBUF):
        if u >= 2:
            _zero(bufs[u])
        _paint(bufs[u], u * CW)
        pltpu.async_copy(bufs[u], out_hbm.at[pl.ds(wbase + u * CW, CW)], sems[u])

    def _group(g, c):
        for u in range(NBUF):
            lo = (g * NBUF + u) * CW
            _drain(bufs[u], sems[u])
            _paint(bufs[u], lo)
            pltpu.async_copy(bufs[u], out_hbm.at[pl.ds(wbase + lo, CW)], sems[u])
        return c

    lax.fori_loop(1, NGRP, _group, 0)

    for u in range(NBUF):
        _drain(bufs[u], sems[u])


def kernel(loc):
    xy = loc.reshape(32, 128, 2).transpose(0, 2, 1).reshape(-1)
    flat = _onehot2d_sc(xy)
    return flat.reshape(BX, BY, B, 1).transpose(2, 0, 1, 3)
